# Initial kernel scaffold; baseline (speedup 1.0000x reference)
import jax
import jax.numpy as jnp
from jax.experimental import pallas as pl


def kernel(x, edge_index, Wl1, bl1, Wr1, Wl2, bl2, Wr2, Wl3, bl3, Wr3, Wfc, bfc):
    raise NotImplementedError("write your pallas kernel here")


# speedup vs baseline: 3.6218x; 3.6218x over previous
import jax
import jax.numpy as jnp
from jax import lax
from jax.experimental import pallas as pl
from jax.experimental.pallas import tpu as pltpu
from jax.experimental.pallas import tpu_sc as plsc

F32 = jnp.float32
W = 128


def _dotT(x, w):
    return lax.dot_general(x, w, (((1,), (1,)), ((), ())),
                           preferred_element_type=F32)


def _pre_body(x_ref, we_ref, be_ref, wr_ref, bl_ref, a_ref, b_ref):
    xb = x_ref[...]
    a_ref[...] = _dotT(xb, we_ref[...]) + be_ref[...]
    b_ref[...] = _dotT(xb, wr_ref[...]) + bl_ref[...]


def _mid_body(s0_ref, s1_ref, b_ref, we_ref, be_ref, wr_ref, bl_ref,
              a_ref, b_out_ref):
    s = s0_ref[0] + s1_ref[0]
    deg = jnp.maximum(s[:, 64:65], 1.0)
    h = jnp.maximum(s[:, :64] / deg + b_ref[...], 0.0)
    a_ref[...] = _dotT(h, we_ref[...]) + be_ref[...]
    b_out_ref[...] = _dotT(h, wr_ref[...]) + bl_ref[...]


def _post_body(s0_ref, s1_ref, b_ref, wfc_ref, bfc_ref, o_ref):
    s = s0_ref[0] + s1_ref[0]
    deg = jnp.maximum(s[:, 64:65], 1.0)
    h = jnp.maximum(s[:, :64] / deg + b_ref[...], 0.0)
    o_ref[...] = _dotT(h, wfc_ref[...]) + bfc_ref[...]


def _row_spec(blk, width):
    return pl.BlockSpec((blk, width), lambda i: (i, 0))


def _full_spec(shape):
    nd = len(shape)
    return pl.BlockSpec(shape, lambda i: (0,) * nd)


def _partial_spec(blk, width, part):
    return pl.BlockSpec((1, blk, width), lambda i, p=part: (p, i, 0))


_CB = 128


def _chunks(total, step):
    plan, off = [], 0
    while off < total:
        sz = min(step, total - off)
        plan.append((off, sz))
        off += sz
    return plan


def _make_sc_segsum(NP, ept_chunks):
    info = plsc.get_sparse_core_info()
    NC, NS = info.num_cores, info.num_subcores
    ZR = NP // NS
    mesh = plsc.VectorSubcoreMesh(core_axis_name="c", subcore_axis_name="s")

    out_type = jax.ShapeDtypeStruct((NC, NP, W), F32)
    scratch = [
        pltpu.VMEM_SHARED((NP, W), F32),
        pltpu.VMEM((_CB,), jnp.int32),
        pltpu.VMEM((_CB,), jnp.int32),
        pltpu.VMEM((_CB, W), F32),
        pltpu.SemaphoreType.DMA,
    ]

    def body(a_hbm, src_hbm, dst_hbm, s_out, s_tab, src_v, dst_v, rows_v,
             sem):
        c = lax.axis_index("c")
        s = lax.axis_index("s")
        wid = s * NC + c
        r0 = s * ZR

        z16 = jnp.zeros((16,), F32)

        def fill_zero(i, carry):
            for j in range(W // 16):
                rows_v[i, pl.ds(j * 16, 16)] = z16
            return carry

        lax.fori_loop(0, _CB, fill_zero, 0)
        for off, sz in _chunks(ZR, _CB):
            pltpu.sync_copy(rows_v.at[pl.ds(0, sz)],
                            s_tab.at[pl.ds(r0 + off, sz)])
        plsc.subcore_barrier()

        def chunk(k, carry):
            base = (wid * ept_chunks + k) * _CB
            pltpu.sync_copy(src_hbm.at[pl.ds(base, _CB)], src_v)
            pltpu.sync_copy(dst_hbm.at[pl.ds(base, _CB)], dst_v)
            pltpu.async_copy(a_hbm.at[src_v], rows_v, sem).wait()
            pltpu.sync_copy(rows_v, s_tab.at[dst_v], add=True)
            return carry

        lax.fori_loop(0, ept_chunks, chunk, 0)
        plsc.subcore_barrier()

        for off, sz in _chunks(ZR, _CB):
            pltpu.sync_copy(s_tab.at[pl.ds(r0 + off, sz)],
                            s_out.at[c, pl.ds(r0 + off, sz)])

    return pl.kernel(body, out_type=out_type, mesh=mesh,
                     scratch_types=scratch)


def kernel(x, edge_index, Wl1, bl1, Wr1, Wl2, bl2, Wr2, Wl3, bl3, Wr3,
           Wfc, bfc):
    N, D = x.shape
    H = Wl1.shape[0]
    OUT = Wfc.shape[0]
    E = edge_index.shape[1]
    NW = 32

    NP = ((N + 1 + 127) // 128) * 128
    ept = -(-E // (NW * _CB))
    EP = NW * ept * _CB
    src = edge_index[0]
    dst = edge_index[1]
    if EP > E:
        src = jnp.pad(src, (0, EP - E))
        dst = jnp.pad(dst, (0, EP - E), constant_values=N)
    xp = jnp.pad(x, ((0, NP - N), (0, 0)))

    def ext(wl):
        return jnp.zeros((W, wl.shape[1]), F32).at[:H].set(wl)

    be = jnp.zeros((1, W), F32).at[0, H].set(1.0)
    We1, We2, We3 = ext(Wl1), ext(Wl2), ext(Wl3)

    BLK = NP // 4
    grid = NP // BLK

    pre = pl.pallas_call(
        _pre_body,
        grid=(grid,),
        in_specs=[_row_spec(BLK, D), _full_spec((W, D)), _full_spec((1, W)),
                  _full_spec((H, D)), _full_spec((1, H))],
        out_specs=[_row_spec(BLK, W), _row_spec(BLK, H)],
        out_shape=[jax.ShapeDtypeStruct((NP, W), F32),
                   jax.ShapeDtypeStruct((NP, H), F32)],
    )
    mid = pl.pallas_call(
        _mid_body,
        grid=(grid,),
        in_specs=[_partial_spec(BLK, W, 0), _partial_spec(BLK, W, 1),
                  _row_spec(BLK, H),
                  _full_spec((W, H)), _full_spec((1, W)),
                  _full_spec((H, H)), _full_spec((1, H))],
        out_specs=[_row_spec(BLK, W), _row_spec(BLK, H)],
        out_shape=[jax.ShapeDtypeStruct((NP, W), F32),
                   jax.ShapeDtypeStruct((NP, H), F32)],
    )
    post = pl.pallas_call(
        _post_body,
        grid=(grid,),
        in_specs=[_partial_spec(BLK, W, 0), _partial_spec(BLK, W, 1),
                  _row_spec(BLK, H),
                  _full_spec((OUT, H)), _full_spec((1, OUT))],
        out_specs=_row_spec(BLK, OUT),
        out_shape=jax.ShapeDtypeStruct((NP, OUT), F32),
    )
    sc_segsum = _make_sc_segsum(NP, ept)

    bl1r = bl1.reshape(1, H)
    bl2r = bl2.reshape(1, H)
    bl3r = bl3.reshape(1, H)
    bfcr = bfc.reshape(1, OUT)

    a1, b1 = pre(xp, We1, be, Wr1, bl1r)
    s1 = sc_segsum(a1, src, dst)
    a2, b2 = mid(s1, s1, b1, We2, be, Wr2, bl2r)
    s2 = sc_segsum(a2, src, dst)
    a3, b3 = mid(s2, s2, b2, We3, be, Wr3, bl3r)
    s3 = sc_segsum(a3, src, dst)
    out = post(s3, s3, b3, Wfc, bfcr)
    return out[:N]
